# Initial kernel scaffold; baseline (speedup 1.0000x reference)
import jax
import jax.numpy as jnp
from jax.experimental import pallas as pl


def kernel(x, edge_index, edge_weights, W1, b1, g1, be1, W2, b2, g2, be2, W3, b3, Wl, bl):
    raise NotImplementedError("write your pallas kernel here")


# speedup vs baseline: 1.0073x; 1.0073x over previous
import jax
import jax.numpy as jnp
from jax.experimental import pallas as pl


def _head(a_ref, w_ref, b_ref, o_ref):
    o_ref[...] = jnp.dot(a_ref[...], w_ref[...],
                         preferred_element_type=jnp.float32) + b_ref[...]


def _gcn_conv(x, src, dst, ew, W, b):
    n = x.shape[0]
    loop = jnp.arange(n, dtype=src.dtype)
    src2 = jnp.concatenate([src, loop])
    dst2 = jnp.concatenate([dst, loop])
    ew2 = jnp.concatenate([ew, jnp.ones((n,), x.dtype)])
    deg = jax.ops.segment_sum(ew2, dst2, num_segments=n)
    dinv = jnp.where(deg > 0, 1.0 / jnp.sqrt(deg), 0.0)
    norm = dinv[src2] * ew2 * dinv[dst2]
    h = x @ W
    msgs = h[src2] * norm[:, None]
    out = jax.ops.segment_sum(msgs, dst2, num_segments=n)
    return out + b


def _batchnorm(h, gamma, beta, eps=1e-5):
    mu = jnp.mean(h, axis=0)
    var = jnp.var(h, axis=0)
    return (h - mu) / jnp.sqrt(var + eps) * gamma + beta


def kernel(x, edge_index, edge_weights, W1, b1, g1, be1, W2, b2, g2, be2, W3, b3, Wl, bl):
    src = edge_index[0]
    dst = edge_index[1]
    out1 = jax.nn.relu(_gcn_conv(x, src, dst, edge_weights, W1, b1))
    out1 = _batchnorm(out1, g1, be1)
    out2 = jax.nn.relu(_gcn_conv(out1, src, dst, edge_weights, W2, b2))
    out2 = _batchnorm(out2, g2, be2)
    out3 = jax.nn.relu(_gcn_conv(out2, src, dst, edge_weights, W3, b3))
    input_lin = jnp.concatenate([out1, out2, out3], axis=1)
    n, c = input_lin.shape[0], Wl.shape[1]
    return pl.pallas_call(
        _head,
        out_shape=jax.ShapeDtypeStruct((n, c), jnp.float32),
    )(input_lin, Wl, bl.reshape(1, c))


# speedup vs baseline: 13.1292x; 13.0341x over previous
import functools

import jax
import jax.numpy as jnp
from jax import lax
from jax.experimental import pallas as pl
from jax.experimental.pallas import tpu as pltpu
from jax.experimental.pallas import tpu_sc as plsc

N = 10000
E = 320000
F_IN = 128
H = 20
HP = 32
C = 10
NC = 2
NS = 16
RPT = 624
TAIL0 = RPT * NS
CK = 80
EPT = E // (NC * NS)
NCK = EPT // CK

_mesh = plsc.VectorSubcoreMesh(core_axis_name="c", subcore_axis_name="s")


def _zero16(ref, nvec):
    def body(i, _):
        ref[pl.ds(i * 16, 16)] = jnp.zeros((16,), jnp.float32)
        return 0
    lax.fori_loop(0, nvec, body, 0)


@functools.partial(
    pl.kernel,
    out_type=[jax.ShapeDtypeStruct((N,), jnp.float32),
              jax.ShapeDtypeStruct((N,), jnp.float32)],
    mesh=_mesh,
    scratch_types=[
        pltpu.VMEM_SHARED((N,), jnp.float32),
        pltpu.VMEM((CK,), jnp.int32),
        pltpu.VMEM((CK,), jnp.float32),
        pltpu.VMEM((640,), jnp.float32),
    ],
    compiler_params=pltpu.CompilerParams(use_tc_tiling_on_sc=False),
)
def _deg(dst_hbm, ew_hbm, out0_hbm, out1_hbm, acc, dst_v, ew_v, zb):
    c = lax.axis_index("c")
    s = lax.axis_index("s")
    r0 = s * RPT
    _zero16(zb, 640 // 16)
    pltpu.sync_copy(zb.at[pl.ds(0, RPT)], acc.at[pl.ds(r0, RPT)])

    @pl.when(s == NS - 1)
    def _():
        pltpu.sync_copy(zb.at[pl.ds(0, N - TAIL0)], acc.at[pl.ds(TAIL0, N - TAIL0)])

    plsc.subcore_barrier()
    e0 = c * (E // NC) + s * EPT

    def chunk(j, _):
        base = e0 + j * CK
        pltpu.sync_copy(dst_hbm.at[pl.ds(base, CK)], dst_v)
        pltpu.sync_copy(ew_hbm.at[pl.ds(base, CK)], ew_v)
        pltpu.sync_copy(ew_v, acc.at[dst_v], add=True)
        return 0

    lax.fori_loop(0, NCK, chunk, 0)
    plsc.subcore_barrier()
    pltpu.sync_copy(acc.at[pl.ds(r0, RPT)], zb.at[pl.ds(0, RPT)])

    @pl.when(c == 0)
    def _():
        pltpu.sync_copy(zb.at[pl.ds(0, RPT)], out0_hbm.at[pl.ds(r0, RPT)])

    @pl.when(c == 1)
    def _():
        pltpu.sync_copy(zb.at[pl.ds(0, RPT)], out1_hbm.at[pl.ds(r0, RPT)])

    @pl.when(s == NS - 1)
    def _():
        pltpu.sync_copy(acc.at[pl.ds(TAIL0, N - TAIL0)], zb.at[pl.ds(0, N - TAIL0)])

        @pl.when(c == 0)
        def _():
            pltpu.sync_copy(zb.at[pl.ds(0, N - TAIL0)],
                            out0_hbm.at[pl.ds(TAIL0, N - TAIL0)])

        @pl.when(c == 1)
        def _():
            pltpu.sync_copy(zb.at[pl.ds(0, N - TAIL0)],
                            out1_hbm.at[pl.ds(TAIL0, N - TAIL0)])


@functools.partial(
    pl.kernel,
    out_type=[jax.ShapeDtypeStruct((N, HP), jnp.float32),
              jax.ShapeDtypeStruct((N, HP), jnp.float32)],
    mesh=_mesh,
    scratch_types=[
        pltpu.VMEM_SHARED((N, HP), jnp.float32),
        pltpu.VMEM_SHARED((N, HP), jnp.float32),
        pltpu.VMEM((CK,), jnp.int32),
        pltpu.VMEM((CK,), jnp.int32),
        pltpu.VMEM((CK,), jnp.float32),
        pltpu.VMEM((CK, HP), jnp.float32),
        pltpu.VMEM((640, HP), jnp.float32),
        pltpu.SemaphoreType.DMA,
    ],
    compiler_params=pltpu.CompilerParams(use_tc_tiling_on_sc=False),
)
def _agg(h_hbm, src_hbm, dst_hbm, ew_hbm, out0_hbm, out1_hbm,
         hS, acc, src_v, dst_v, ew_v, rows_v, bounce, sem):
    c = lax.axis_index("c")
    s = lax.axis_index("s")
    r0 = s * RPT
    pltpu.sync_copy(h_hbm.at[pl.ds(r0, RPT)], bounce.at[pl.ds(0, RPT)])
    pltpu.sync_copy(bounce.at[pl.ds(0, RPT)], hS.at[pl.ds(r0, RPT)])

    @pl.when(s == NS - 1)
    def _():
        pltpu.sync_copy(h_hbm.at[pl.ds(TAIL0, N - TAIL0)],
                        bounce.at[pl.ds(RPT, N - TAIL0)])
        pltpu.sync_copy(bounce.at[pl.ds(RPT, N - TAIL0)],
                        hS.at[pl.ds(TAIL0, N - TAIL0)])

    def zr(i, _):
        rows_v[i, 0:16] = jnp.zeros((16,), jnp.float32)
        rows_v[i, 16:32] = jnp.zeros((16,), jnp.float32)
        return 0
    lax.fori_loop(0, CK, zr, 0)

    def zc(i, _):
        pltpu.sync_copy(rows_v, acc.at[pl.ds(r0 + i * CK, CK)])
        return 0
    lax.fori_loop(0, RPT // CK, zc, 0)
    pltpu.sync_copy(rows_v.at[pl.ds(0, RPT % CK)],
                    acc.at[pl.ds(r0 + CK * (RPT // CK), RPT % CK)])

    @pl.when(s == NS - 1)
    def _():
        pltpu.sync_copy(rows_v.at[pl.ds(0, N - TAIL0)],
                        acc.at[pl.ds(TAIL0, N - TAIL0)])

    plsc.subcore_barrier()
    e0 = c * (E // NC) + s * EPT

    def chunk(j, _):
        base = e0 + j * CK
        pltpu.sync_copy(src_hbm.at[pl.ds(base, CK)], src_v)
        pltpu.sync_copy(dst_hbm.at[pl.ds(base, CK)], dst_v)
        pltpu.sync_copy(ew_hbm.at[pl.ds(base, CK)], ew_v)
        pltpu.async_copy(hS.at[src_v], rows_v, sem).wait()

        def scale(g, _):
            k0 = g * 16
            nv = ew_v[pl.ds(k0, 16)]
            for l in range(16):
                w = nv[l]
                rows_v[k0 + l, 0:16] = rows_v[k0 + l, 0:16] * w
                rows_v[k0 + l, 16:32] = rows_v[k0 + l, 16:32] * w
            return 0
        lax.fori_loop(0, CK // 16, scale, 0)
        pltpu.sync_copy(rows_v, acc.at[dst_v], add=True)
        return 0

    lax.fori_loop(0, NCK, chunk, 0)
    plsc.subcore_barrier()
    pltpu.sync_copy(acc.at[pl.ds(r0, RPT)], bounce.at[pl.ds(0, RPT)])

    @pl.when(s == NS - 1)
    def _():
        pltpu.sync_copy(acc.at[pl.ds(TAIL0, N - TAIL0)],
                        bounce.at[pl.ds(RPT, N - TAIL0)])

    @pl.when(c == 0)
    def _():
        pltpu.sync_copy(bounce.at[pl.ds(0, RPT)], out0_hbm.at[pl.ds(r0, RPT)])

    @pl.when(c == 1)
    def _():
        pltpu.sync_copy(bounce.at[pl.ds(0, RPT)], out1_hbm.at[pl.ds(r0, RPT)])

    @pl.when(s == NS - 1)
    def _():
        @pl.when(c == 0)
        def _():
            pltpu.sync_copy(bounce.at[pl.ds(RPT, N - TAIL0)],
                            out0_hbm.at[pl.ds(TAIL0, N - TAIL0)])

        @pl.when(c == 1)
        def _():
            pltpu.sync_copy(bounce.at[pl.ds(RPT, N - TAIL0)],
                            out1_hbm.at[pl.ds(TAIL0, N - TAIL0)])


def _tc_pre_body(dega_ref, degb_ref, x_ref, w1_ref, dinv_ref, h1_ref):
    deg = dega_ref[...] + degb_ref[...] + 1.0
    dinv = lax.rsqrt(deg)
    dinv_ref[...] = dinv
    h = jnp.dot(x_ref[...], w1_ref[...], preferred_element_type=jnp.float32)
    h1_ref[...] = h * dinv


def _tc_mid_body(agga_ref, aggb_ref, hp_ref, dinv_ref, b_ref, g_ref, be_ref,
                 wn_ref, bn_ref, hn_ref):
    dinv = dinv_ref[...]
    pre = (agga_ref[...] + aggb_ref[...] + hp_ref[...]) * dinv + b_ref[...]
    r = jnp.maximum(pre, 0.0)
    mu = jnp.mean(r, axis=0, keepdims=True)
    var = jnp.mean((r - mu) * (r - mu), axis=0, keepdims=True)
    bn = (r - mu) / jnp.sqrt(var + 1e-5) * g_ref[...] + be_ref[...]
    bn_ref[...] = bn
    hn_ref[...] = jnp.dot(bn, wn_ref[...], preferred_element_type=jnp.float32) * dinv


def _tc_head_body(agga_ref, aggb_ref, hp_ref, dinv_ref, b_ref,
                  o1_ref, o2_ref, wl1_ref, wl2_ref, wl3_ref, bl_ref, y_ref):
    pre = (agga_ref[...] + aggb_ref[...] + hp_ref[...]) * dinv_ref[...] + b_ref[...]
    out3 = jnp.maximum(pre, 0.0)
    y = (jnp.dot(o1_ref[...], wl1_ref[...], preferred_element_type=jnp.float32)
         + jnp.dot(o2_ref[...], wl2_ref[...], preferred_element_type=jnp.float32)
         + jnp.dot(out3, wl3_ref[...], preferred_element_type=jnp.float32))
    y_ref[...] = y + bl_ref[...]


def kernel(x, edge_index, edge_weights, W1, b1, g1, be1, W2, b2, g2, be2,
           W3, b3, Wl, bl):
    src = edge_index[0]
    dst = edge_index[1]
    padf = lambda v: jnp.pad(v, (0, HP - H)).reshape(1, HP)
    W1p = jnp.pad(W1, ((0, 0), (0, HP - H)))
    W2p = jnp.pad(W2, ((0, HP - H), (0, HP - H)))
    W3p = jnp.pad(W3, ((0, HP - H), (0, HP - H)))
    Wl1 = jnp.pad(Wl[0:H], ((0, HP - H), (0, 0)))
    Wl2 = jnp.pad(Wl[H:2 * H], ((0, HP - H), (0, 0)))
    Wl3 = jnp.pad(Wl[2 * H:3 * H], ((0, HP - H), (0, 0)))

    deg0, deg1 = _deg(dst, edge_weights)
    dega = deg0.reshape(N, 1)
    degb = deg1.reshape(N, 1)

    dinv, h1p = pl.pallas_call(
        _tc_pre_body,
        out_shape=[jax.ShapeDtypeStruct((N, 1), jnp.float32),
                   jax.ShapeDtypeStruct((N, HP), jnp.float32)],
    )(dega, degb, x, W1p)

    agg1 = _agg(h1p, src, dst, edge_weights)
    out1, h2p = pl.pallas_call(
        _tc_mid_body,
        out_shape=[jax.ShapeDtypeStruct((N, HP), jnp.float32),
                   jax.ShapeDtypeStruct((N, HP), jnp.float32)],
    )(agg1[0], agg1[1], h1p, dinv, padf(b1), padf(g1), padf(be1), W2p)

    agg2 = _agg(h2p, src, dst, edge_weights)
    out2, h3p = pl.pallas_call(
        _tc_mid_body,
        out_shape=[jax.ShapeDtypeStruct((N, HP), jnp.float32),
                   jax.ShapeDtypeStruct((N, HP), jnp.float32)],
    )(agg2[0], agg2[1], h2p, dinv, padf(b2), padf(g2), padf(be2), W3p)

    agg3 = _agg(h3p, src, dst, edge_weights)
    y = pl.pallas_call(
        _tc_head_body,
        out_shape=jax.ShapeDtypeStruct((N, C), jnp.float32),
    )(agg3[0], agg3[1], h3p, dinv, padf(b3), out1, out2, Wl1, Wl2, Wl3,
      bl.reshape(1, C))
    return y


# speedup vs baseline: 40.4720x; 3.0826x over previous
import functools

import jax
import jax.numpy as jnp
from jax import lax
from jax.experimental import pallas as pl
from jax.experimental.pallas import tpu as pltpu
from jax.experimental.pallas import tpu_sc as plsc

N = 10000
E = 320000
F_IN = 128
H = 20
HP = 32
C = 10
NC = 2
NS = 16
RPT = 624
TAIL0 = RPT * NS
CK = 80
EPT = E // (NC * NS)
NCK = EPT // CK

_mesh = plsc.VectorSubcoreMesh(core_axis_name="c", subcore_axis_name="s")
_sc_params = pltpu.CompilerParams(use_tc_tiling_on_sc=False)


def _zero16(ref, nvec):
    def body(i, _):
        ref[pl.ds(i * 16, 16)] = jnp.zeros((16,), jnp.float32)
        return 0
    lax.fori_loop(0, nvec, body, 0)


def _refill(dstv, dst_flat, j):
    for k0 in range(0, CK, 16):
        dstv[pl.ds(k0, 16)] = dst_flat[pl.ds(j * CK + k0, 16)]


@functools.partial(
    pl.kernel,
    out_type=[jax.ShapeDtypeStruct((N,), jnp.float32),
              jax.ShapeDtypeStruct((N,), jnp.float32)],
    mesh=_mesh,
    scratch_types=[
        pltpu.VMEM_SHARED((N,), jnp.float32),
        pltpu.VMEM((EPT,), jnp.int32),
        pltpu.VMEM((EPT,), jnp.float32),
        pltpu.VMEM((CK,), jnp.int32),
        pltpu.VMEM((640,), jnp.float32),
    ],
    compiler_params=_sc_params,
)
def _deg(dst_hbm, ew_hbm, out0_hbm, out1_hbm, acc, dst_flat, ew_flat, dstv, zb):
    c = lax.axis_index("c")
    s = lax.axis_index("s")
    r0 = s * RPT
    e0 = c * (E // NC) + s * EPT
    pltpu.sync_copy(dst_hbm.at[pl.ds(e0, EPT)], dst_flat)
    pltpu.sync_copy(ew_hbm.at[pl.ds(e0, EPT)], ew_flat)
    _zero16(zb, 640 // 16)
    pltpu.sync_copy(zb.at[pl.ds(0, RPT)], acc.at[pl.ds(r0, RPT)])

    @pl.when(s == NS - 1)
    def _():
        pltpu.sync_copy(zb.at[pl.ds(0, N - TAIL0)], acc.at[pl.ds(TAIL0, N - TAIL0)])

    plsc.subcore_barrier()

    def chunk(j, _):
        _refill(dstv, dst_flat, j)
        pltpu.sync_copy(ew_flat.at[pl.ds(j * CK, CK)], acc.at[dstv], add=True)
        return 0

    lax.fori_loop(0, NCK, chunk, 0)
    plsc.subcore_barrier()
    pltpu.sync_copy(acc.at[pl.ds(r0, RPT)], zb.at[pl.ds(0, RPT)])

    @pl.when(c == 0)
    def _():
        pltpu.sync_copy(zb.at[pl.ds(0, RPT)], out0_hbm.at[pl.ds(r0, RPT)])

    @pl.when(c == 1)
    def _():
        pltpu.sync_copy(zb.at[pl.ds(0, RPT)], out1_hbm.at[pl.ds(r0, RPT)])

    @pl.when(s == NS - 1)
    def _():
        pltpu.sync_copy(acc.at[pl.ds(TAIL0, N - TAIL0)], zb.at[pl.ds(0, N - TAIL0)])

        @pl.when(c == 0)
        def _():
            pltpu.sync_copy(zb.at[pl.ds(0, N - TAIL0)],
                            out0_hbm.at[pl.ds(TAIL0, N - TAIL0)])

        @pl.when(c == 1)
        def _():
            pltpu.sync_copy(zb.at[pl.ds(0, N - TAIL0)],
                            out1_hbm.at[pl.ds(TAIL0, N - TAIL0)])


@functools.partial(
    pl.kernel,
    out_type=[jax.ShapeDtypeStruct((N, HP), jnp.float32),
              jax.ShapeDtypeStruct((N, HP), jnp.float32)],
    mesh=_mesh,
    scratch_types=[
        pltpu.VMEM_SHARED((N, HP), jnp.float32),
        pltpu.VMEM_SHARED((N, HP), jnp.float32),
        pltpu.VMEM((EPT,), jnp.int32),
        pltpu.VMEM((EPT,), jnp.int32),
        pltpu.VMEM((EPT,), jnp.float32),
        pltpu.VMEM((CK, HP), jnp.float32),
        pltpu.VMEM((CK, HP), jnp.float32),
        pltpu.VMEM((CK,), jnp.int32),
        pltpu.VMEM((CK,), jnp.int32),
        pltpu.VMEM((640, HP), jnp.float32),
        pltpu.SemaphoreType.DMA,
        pltpu.SemaphoreType.DMA,
    ],
    compiler_params=_sc_params,
)
def _agg(h_hbm, src_hbm, dst_hbm, ew_hbm, out0_hbm, out1_hbm,
         hS, acc, src_flat, dst_flat, ew_flat,
         rows_a, rows_b, dstv_a, dstv_b, bounce, gsem_a, gsem_b):
    c = lax.axis_index("c")
    s = lax.axis_index("s")
    r0 = s * RPT
    e0 = c * (E // NC) + s * EPT
    pltpu.sync_copy(src_hbm.at[pl.ds(e0, EPT)], src_flat)
    pltpu.sync_copy(dst_hbm.at[pl.ds(e0, EPT)], dst_flat)
    pltpu.sync_copy(ew_hbm.at[pl.ds(e0, EPT)], ew_flat)
    pltpu.sync_copy(h_hbm.at[pl.ds(r0, RPT)], bounce.at[pl.ds(0, RPT)])
    pltpu.sync_copy(bounce.at[pl.ds(0, RPT)], hS.at[pl.ds(r0, RPT)])

    @pl.when(s == NS - 1)
    def _():
        pltpu.sync_copy(h_hbm.at[pl.ds(TAIL0, N - TAIL0)],
                        bounce.at[pl.ds(RPT, N - TAIL0)])
        pltpu.sync_copy(bounce.at[pl.ds(RPT, N - TAIL0)],
                        hS.at[pl.ds(TAIL0, N - TAIL0)])

    def zr(i, _):
        rows_a[i, 0:16] = jnp.zeros((16,), jnp.float32)
        rows_a[i, 16:32] = jnp.zeros((16,), jnp.float32)
        return 0
    lax.fori_loop(0, CK, zr, 0)

    def zc(i, _):
        pltpu.sync_copy(rows_a, acc.at[pl.ds(r0 + i * CK, CK)])
        return 0
    lax.fori_loop(0, RPT // CK, zc, 0)
    pltpu.sync_copy(rows_a.at[pl.ds(0, RPT % CK)],
                    acc.at[pl.ds(r0 + CK * (RPT // CK), RPT % CK)])

    @pl.when(s == NS - 1)
    def _():
        pltpu.sync_copy(rows_a.at[pl.ds(0, N - TAIL0)],
                        acc.at[pl.ds(TAIL0, N - TAIL0)])

    plsc.subcore_barrier()

    def gather(j, rows, sem):
        return pltpu.async_copy(hS.at[src_flat.at[pl.ds(j * CK, CK)]], rows, sem)

    def gather_wait(j, rows, sem):
        pltpu.make_async_copy(hS.at[src_flat.at[pl.ds(j * CK, CK)]], rows, sem).wait()

    def scale(rows, j):
        def grp(g, _):
            k0 = g * 16
            nv = ew_flat[pl.ds(j * CK + k0, 16)]
            for l in range(16):
                w = nv[l]
                rows[k0 + l, 0:16] = rows[k0 + l, 0:16] * w
                rows[k0 + l, 16:32] = rows[k0 + l, 16:32] * w
            return 0
        lax.fori_loop(0, CK // 16, grp, 0)

    gather(0, rows_a, gsem_a)

    def pair(jj, _):
        a = 2 * jj
        b = a + 1
        gather_wait(a, rows_a, gsem_a)
        gather(b, rows_b, gsem_b)
        scale(rows_a, a)
        _refill(dstv_a, dst_flat, a)
        pltpu.sync_copy(rows_a, acc.at[dstv_a], add=True)
        gather_wait(b, rows_b, gsem_b)
        gather(a + 2, rows_a, gsem_a)
        scale(rows_b, b)
        _refill(dstv_b, dst_flat, b)
        pltpu.sync_copy(rows_b, acc.at[dstv_b], add=True)
        return 0

    lax.fori_loop(0, (NCK - 1) // 2, pair, 0)
    last = NCK - 1
    gather_wait(last, rows_a, gsem_a)
    scale(rows_a, last)
    _refill(dstv_a, dst_flat, last)
    pltpu.sync_copy(rows_a, acc.at[dstv_a], add=True)

    plsc.subcore_barrier()
    pltpu.sync_copy(acc.at[pl.ds(r0, RPT)], bounce.at[pl.ds(0, RPT)])

    @pl.when(s == NS - 1)
    def _():
        pltpu.sync_copy(acc.at[pl.ds(TAIL0, N - TAIL0)],
                        bounce.at[pl.ds(RPT, N - TAIL0)])

    @pl.when(c == 0)
    def _():
        pltpu.sync_copy(bounce.at[pl.ds(0, RPT)], out0_hbm.at[pl.ds(r0, RPT)])

    @pl.when(c == 1)
    def _():
        pltpu.sync_copy(bounce.at[pl.ds(0, RPT)], out1_hbm.at[pl.ds(r0, RPT)])

    @pl.when(s == NS - 1)
    def _():
        @pl.when(c == 0)
        def _():
            pltpu.sync_copy(bounce.at[pl.ds(RPT, N - TAIL0)],
                            out0_hbm.at[pl.ds(TAIL0, N - TAIL0)])

        @pl.when(c == 1)
        def _():
            pltpu.sync_copy(bounce.at[pl.ds(RPT, N - TAIL0)],
                            out1_hbm.at[pl.ds(TAIL0, N - TAIL0)])


def _tc_pre_body(dega_ref, degb_ref, x_ref, w1_ref, dinv_ref, h1_ref):
    deg = dega_ref[...] + degb_ref[...] + 1.0
    dinv = lax.rsqrt(deg)
    dinv_ref[...] = dinv
    h = jnp.dot(x_ref[...], w1_ref[...], preferred_element_type=jnp.float32)
    h1_ref[...] = h * dinv


def _tc_mid_body(agga_ref, aggb_ref, hp_ref, dinv_ref, b_ref, g_ref, be_ref,
                 wn_ref, bn_ref, hn_ref):
    dinv = dinv_ref[...]
    pre = (agga_ref[...] + aggb_ref[...] + hp_ref[...]) * dinv + b_ref[...]
    r = jnp.maximum(pre, 0.0)
    mu = jnp.mean(r, axis=0, keepdims=True)
    var = jnp.mean((r - mu) * (r - mu), axis=0, keepdims=True)
    bn = (r - mu) / jnp.sqrt(var + 1e-5) * g_ref[...] + be_ref[...]
    bn_ref[...] = bn
    hn_ref[...] = jnp.dot(bn, wn_ref[...], preferred_element_type=jnp.float32) * dinv


def _tc_head_body(agga_ref, aggb_ref, hp_ref, dinv_ref, b_ref,
                  o1_ref, o2_ref, wl1_ref, wl2_ref, wl3_ref, bl_ref, y_ref):
    pre = (agga_ref[...] + aggb_ref[...] + hp_ref[...]) * dinv_ref[...] + b_ref[...]
    out3 = jnp.maximum(pre, 0.0)
    y = (jnp.dot(o1_ref[...], wl1_ref[...], preferred_element_type=jnp.float32)
         + jnp.dot(o2_ref[...], wl2_ref[...], preferred_element_type=jnp.float32)
         + jnp.dot(out3, wl3_ref[...], preferred_element_type=jnp.float32))
    y_ref[...] = y + bl_ref[...]


def kernel(x, edge_index, edge_weights, W1, b1, g1, be1, W2, b2, g2, be2,
           W3, b3, Wl, bl):
    src = edge_index[0]
    dst = edge_index[1]
    padf = lambda v: jnp.pad(v, (0, HP - H)).reshape(1, HP)
    W1p = jnp.pad(W1, ((0, 0), (0, HP - H)))
    W2p = jnp.pad(W2, ((0, HP - H), (0, HP - H)))
    W3p = jnp.pad(W3, ((0, HP - H), (0, HP - H)))
    Wl1 = jnp.pad(Wl[0:H], ((0, HP - H), (0, 0)))
    Wl2 = jnp.pad(Wl[H:2 * H], ((0, HP - H), (0, 0)))
    Wl3 = jnp.pad(Wl[2 * H:3 * H], ((0, HP - H), (0, 0)))

    deg0, deg1 = _deg(dst, edge_weights)
    dega = deg0.reshape(N, 1)
    degb = deg1.reshape(N, 1)

    dinv, h1p = pl.pallas_call(
        _tc_pre_body,
        out_shape=[jax.ShapeDtypeStruct((N, 1), jnp.float32),
                   jax.ShapeDtypeStruct((N, HP), jnp.float32)],
    )(dega, degb, x, W1p)

    agg1 = _agg(h1p, src, dst, edge_weights)
    out1, h2p = pl.pallas_call(
        _tc_mid_body,
        out_shape=[jax.ShapeDtypeStruct((N, HP), jnp.float32),
                   jax.ShapeDtypeStruct((N, HP), jnp.float32)],
    )(agg1[0], agg1[1], h1p, dinv, padf(b1), padf(g1), padf(be1), W2p)

    agg2 = _agg(h2p, src, dst, edge_weights)
    out2, h3p = pl.pallas_call(
        _tc_mid_body,
        out_shape=[jax.ShapeDtypeStruct((N, HP), jnp.float32),
                   jax.ShapeDtypeStruct((N, HP), jnp.float32)],
    )(agg2[0], agg2[1], h2p, dinv, padf(b2), padf(g2), padf(be2), W3p)

    agg3 = _agg(h3p, src, dst, edge_weights)
    y = pl.pallas_call(
        _tc_head_body,
        out_shape=jax.ShapeDtypeStruct((N, C), jnp.float32),
    )(agg3[0], agg3[1], h3p, dinv, padf(b3), out1, out2, Wl1, Wl2, Wl3,
      bl.reshape(1, C))
    return y


# speedup vs baseline: 42.5986x; 1.0525x over previous
import functools

import jax
import jax.numpy as jnp
from jax import lax
from jax.experimental import pallas as pl
from jax.experimental.pallas import tpu as pltpu
from jax.experimental.pallas import tpu_sc as plsc

N = 10000
E = 320000
F_IN = 128
H = 20
HP = 32
C = 10
NC = 2
NS = 16
RPT = 624
TAIL0 = RPT * NS
CK = 80
EPT = E // (NC * NS)
NCK = EPT // CK

_mesh = plsc.VectorSubcoreMesh(core_axis_name="c", subcore_axis_name="s")
_sc_params = pltpu.CompilerParams(use_tc_tiling_on_sc=False)


def _zero16(ref, nvec):
    def body(i, _):
        ref[pl.ds(i * 16, 16)] = jnp.zeros((16,), jnp.float32)
        return 0
    lax.fori_loop(0, nvec, body, 0)


def _refill(dstv, dst_flat, j):
    for k0 in range(0, CK, 16):
        dstv[pl.ds(k0, 16)] = dst_flat[pl.ds(j * CK + k0, 16)]


@functools.partial(
    pl.kernel,
    out_type=[jax.ShapeDtypeStruct((N,), jnp.float32),
              jax.ShapeDtypeStruct((N,), jnp.float32)],
    mesh=_mesh,
    scratch_types=[
        pltpu.VMEM_SHARED((N,), jnp.float32),
        pltpu.VMEM((EPT,), jnp.int32),
        pltpu.VMEM((EPT,), jnp.float32),
        pltpu.VMEM((CK,), jnp.int32),
        pltpu.VMEM((640,), jnp.float32),
    ],
    compiler_params=_sc_params,
)
def _deg(dst_hbm, ew_hbm, out0_hbm, out1_hbm, acc, dst_flat, ew_flat, dstv, zb):
    c = lax.axis_index("c")
    s = lax.axis_index("s")
    r0 = s * RPT
    e0 = c * (E // NC) + s * EPT
    pltpu.sync_copy(dst_hbm.at[pl.ds(e0, EPT)], dst_flat)
    pltpu.sync_copy(ew_hbm.at[pl.ds(e0, EPT)], ew_flat)
    _zero16(zb, 640 // 16)
    pltpu.sync_copy(zb.at[pl.ds(0, RPT)], acc.at[pl.ds(r0, RPT)])

    @pl.when(s == NS - 1)
    def _():
        pltpu.sync_copy(zb.at[pl.ds(0, N - TAIL0)], acc.at[pl.ds(TAIL0, N - TAIL0)])

    plsc.subcore_barrier()

    def chunk(j, _):
        _refill(dstv, dst_flat, j)
        pltpu.sync_copy(ew_flat.at[pl.ds(j * CK, CK)], acc.at[dstv], add=True)
        return 0

    lax.fori_loop(0, NCK, chunk, 0)
    plsc.subcore_barrier()
    pltpu.sync_copy(acc.at[pl.ds(r0, RPT)], zb.at[pl.ds(0, RPT)])

    @pl.when(c == 0)
    def _():
        pltpu.sync_copy(zb.at[pl.ds(0, RPT)], out0_hbm.at[pl.ds(r0, RPT)])

    @pl.when(c == 1)
    def _():
        pltpu.sync_copy(zb.at[pl.ds(0, RPT)], out1_hbm.at[pl.ds(r0, RPT)])

    @pl.when(s == NS - 1)
    def _():
        pltpu.sync_copy(acc.at[pl.ds(TAIL0, N - TAIL0)], zb.at[pl.ds(0, N - TAIL0)])

        @pl.when(c == 0)
        def _():
            pltpu.sync_copy(zb.at[pl.ds(0, N - TAIL0)],
                            out0_hbm.at[pl.ds(TAIL0, N - TAIL0)])

        @pl.when(c == 1)
        def _():
            pltpu.sync_copy(zb.at[pl.ds(0, N - TAIL0)],
                            out1_hbm.at[pl.ds(TAIL0, N - TAIL0)])


@functools.partial(
    pl.kernel,
    out_type=[jax.ShapeDtypeStruct((N, HP), jnp.float32),
              jax.ShapeDtypeStruct((N, HP), jnp.float32)],
    mesh=_mesh,
    scratch_types=[
        pltpu.VMEM_SHARED((N, HP), jnp.float32),
        pltpu.VMEM_SHARED((N, HP), jnp.float32),
        pltpu.VMEM((EPT,), jnp.int32),
        pltpu.VMEM((EPT,), jnp.int32),
        pltpu.VMEM((EPT,), jnp.float32),
        pltpu.VMEM((CK, HP), jnp.float32),
        pltpu.VMEM((CK, HP), jnp.float32),
        pltpu.VMEM((CK, HP), jnp.float32),
        pltpu.VMEM((CK,), jnp.int32),
        pltpu.VMEM((CK,), jnp.int32),
        pltpu.VMEM((CK,), jnp.int32),
        pltpu.VMEM((640, HP), jnp.float32),
        pltpu.SemaphoreType.DMA,
        pltpu.SemaphoreType.DMA,
        pltpu.SemaphoreType.DMA,
        pltpu.SemaphoreType.DMA,
        pltpu.SemaphoreType.DMA,
        pltpu.SemaphoreType.DMA,
    ],
    compiler_params=_sc_params,
)
def _agg(h_hbm, src_hbm, dst_hbm, ew_hbm, out0_hbm, out1_hbm,
         hS, acc, src_flat, dst_flat, ew_flat,
         rows_0, rows_1, rows_2, dstv_0, dstv_1, dstv_2, bounce,
         gsem_0, gsem_1, gsem_2, ssem_0, ssem_1, ssem_2):
    c = lax.axis_index("c")
    s = lax.axis_index("s")
    r0 = s * RPT
    e0 = c * (E // NC) + s * EPT
    pltpu.sync_copy(src_hbm.at[pl.ds(e0, EPT)], src_flat)
    pltpu.sync_copy(dst_hbm.at[pl.ds(e0, EPT)], dst_flat)
    pltpu.sync_copy(ew_hbm.at[pl.ds(e0, EPT)], ew_flat)
    pltpu.sync_copy(h_hbm.at[pl.ds(r0, RPT)], bounce.at[pl.ds(0, RPT)])
    pltpu.sync_copy(bounce.at[pl.ds(0, RPT)], hS.at[pl.ds(r0, RPT)])

    @pl.when(s == NS - 1)
    def _():
        pltpu.sync_copy(h_hbm.at[pl.ds(TAIL0, N - TAIL0)],
                        bounce.at[pl.ds(RPT, N - TAIL0)])
        pltpu.sync_copy(bounce.at[pl.ds(RPT, N - TAIL0)],
                        hS.at[pl.ds(TAIL0, N - TAIL0)])

    def zr(i, _):
        rows_0[i, 0:16] = jnp.zeros((16,), jnp.float32)
        rows_0[i, 16:32] = jnp.zeros((16,), jnp.float32)
        return 0
    lax.fori_loop(0, CK, zr, 0)

    def zc(i, _):
        pltpu.sync_copy(rows_0, acc.at[pl.ds(r0 + i * CK, CK)])
        return 0
    lax.fori_loop(0, RPT // CK, zc, 0)
    pltpu.sync_copy(rows_0.at[pl.ds(0, RPT % CK)],
                    acc.at[pl.ds(r0 + CK * (RPT // CK), RPT % CK)])

    @pl.when(s == NS - 1)
    def _():
        pltpu.sync_copy(rows_0.at[pl.ds(0, N - TAIL0)],
                        acc.at[pl.ds(TAIL0, N - TAIL0)])

    plsc.subcore_barrier()

    bufs = ((rows_0, dstv_0, gsem_0, ssem_0),
            (rows_1, dstv_1, gsem_1, ssem_1),
            (rows_2, dstv_2, gsem_2, ssem_2))

    def gather(j, rows, sem):
        pltpu.async_copy(hS.at[src_flat.at[pl.ds(j * CK, CK)]], rows, sem)

    def gather_wait(j, rows, sem):
        pltpu.make_async_copy(hS.at[src_flat.at[pl.ds(j * CK, CK)]], rows, sem).wait()

    def scatter_wait(rows, dstv, sem):
        pltpu.make_async_copy(rows, acc.at[dstv], sem).wait()

    def scale(rows, j):
        def grp(g, _):
            k0 = g * 16
            nv = ew_flat[pl.ds(j * CK + k0, 16)]
            for l in range(16):
                w = nv[l]
                rows[k0 + l, 0:16] = rows[k0 + l, 0:16] * w
                rows[k0 + l, 16:32] = rows[k0 + l, 16:32] * w
            return 0
        lax.fori_loop(0, CK // 16, grp, 0)

    def step(j, u, first=False, guard_next=False):
        nu = (u + 1) % 3
        rows, dstv, gsem, ssem = bufs[u]
        nrows, ndstv, ngsem, nssem = bufs[nu]
        if not first:
            scatter_wait(nrows, ndstv, nssem)
        if guard_next:
            @pl.when(j < NCK - 1)
            def _():
                gather(j + 1, nrows, ngsem)
        else:
            gather(j + 1, nrows, ngsem)
        gather_wait(j, rows, gsem)
        scale(rows, j)
        _refill(dstv, dst_flat, j)
        pltpu.async_copy(rows, acc.at[dstv], ssem, add=True)

    gather(0, rows_0, gsem_0)
    step(0, 0, first=True)
    step(1, 1, first=True)

    def tri(t, _):
        j = 3 * t + 2
        step(j, 2, guard_next=True)
        step(j + 1, 0, guard_next=True)
        step(j + 2, 1, guard_next=True)
        return 0

    lax.fori_loop(0, (NCK - 2) // 3, tri, 0)
    scatter_wait(rows_0, dstv_0, ssem_0)
    scatter_wait(rows_1, dstv_1, ssem_1)

    plsc.subcore_barrier()
    pltpu.sync_copy(acc.at[pl.ds(r0, RPT)], bounce.at[pl.ds(0, RPT)])

    @pl.when(s == NS - 1)
    def _():
        pltpu.sync_copy(acc.at[pl.ds(TAIL0, N - TAIL0)],
                        bounce.at[pl.ds(RPT, N - TAIL0)])

    @pl.when(c == 0)
    def _():
        pltpu.sync_copy(bounce.at[pl.ds(0, RPT)], out0_hbm.at[pl.ds(r0, RPT)])

    @pl.when(c == 1)
    def _():
        pltpu.sync_copy(bounce.at[pl.ds(0, RPT)], out1_hbm.at[pl.ds(r0, RPT)])

    @pl.when(s == NS - 1)
    def _():
        @pl.when(c == 0)
        def _():
            pltpu.sync_copy(bounce.at[pl.ds(RPT, N - TAIL0)],
                            out0_hbm.at[pl.ds(TAIL0, N - TAIL0)])

        @pl.when(c == 1)
        def _():
            pltpu.sync_copy(bounce.at[pl.ds(RPT, N - TAIL0)],
                            out1_hbm.at[pl.ds(TAIL0, N - TAIL0)])


def _tc_pre_body(dega_ref, degb_ref, x_ref, w1_ref, dinv_ref, h1_ref):
    deg = dega_ref[...] + degb_ref[...] + 1.0
    dinv = lax.rsqrt(deg)
    dinv_ref[...] = dinv
    h = jnp.dot(x_ref[...], w1_ref[...], preferred_element_type=jnp.float32)
    h1_ref[...] = h * dinv


def _tc_mid_body(agga_ref, aggb_ref, hp_ref, dinv_ref, b_ref, g_ref, be_ref,
                 wn_ref, bn_ref, hn_ref):
    dinv = dinv_ref[...]
    pre = (agga_ref[...] + aggb_ref[...] + hp_ref[...]) * dinv + b_ref[...]
    r = jnp.maximum(pre, 0.0)
    mu = jnp.mean(r, axis=0, keepdims=True)
    var = jnp.mean((r - mu) * (r - mu), axis=0, keepdims=True)
    bn = (r - mu) / jnp.sqrt(var + 1e-5) * g_ref[...] + be_ref[...]
    bn_ref[...] = bn
    hn_ref[...] = jnp.dot(bn, wn_ref[...], preferred_element_type=jnp.float32) * dinv


def _tc_head_body(agga_ref, aggb_ref, hp_ref, dinv_ref, b_ref,
                  o1_ref, o2_ref, wl1_ref, wl2_ref, wl3_ref, bl_ref, y_ref):
    pre = (agga_ref[...] + aggb_ref[...] + hp_ref[...]) * dinv_ref[...] + b_ref[...]
    out3 = jnp.maximum(pre, 0.0)
    y = (jnp.dot(o1_ref[...], wl1_ref[...], preferred_element_type=jnp.float32)
         + jnp.dot(o2_ref[...], wl2_ref[...], preferred_element_type=jnp.float32)
         + jnp.dot(out3, wl3_ref[...], preferred_element_type=jnp.float32))
    y_ref[...] = y + bl_ref[...]


def kernel(x, edge_index, edge_weights, W1, b1, g1, be1, W2, b2, g2, be2,
           W3, b3, Wl, bl):
    src = edge_index[0]
    dst = edge_index[1]
    padf = lambda v: jnp.pad(v, (0, HP - H)).reshape(1, HP)
    W1p = jnp.pad(W1, ((0, 0), (0, HP - H)))
    W2p = jnp.pad(W2, ((0, HP - H), (0, HP - H)))
    W3p = jnp.pad(W3, ((0, HP - H), (0, HP - H)))
    Wl1 = jnp.pad(Wl[0:H], ((0, HP - H), (0, 0)))
    Wl2 = jnp.pad(Wl[H:2 * H], ((0, HP - H), (0, 0)))
    Wl3 = jnp.pad(Wl[2 * H:3 * H], ((0, HP - H), (0, 0)))

    deg0, deg1 = _deg(dst, edge_weights)
    dega = deg0.reshape(N, 1)
    degb = deg1.reshape(N, 1)

    dinv, h1p = pl.pallas_call(
        _tc_pre_body,
        out_shape=[jax.ShapeDtypeStruct((N, 1), jnp.float32),
                   jax.ShapeDtypeStruct((N, HP), jnp.float32)],
    )(dega, degb, x, W1p)

    agg1 = _agg(h1p, src, dst, edge_weights)
    out1, h2p = pl.pallas_call(
        _tc_mid_body,
        out_shape=[jax.ShapeDtypeStruct((N, HP), jnp.float32),
                   jax.ShapeDtypeStruct((N, HP), jnp.float32)],
    )(agg1[0], agg1[1], h1p, dinv, padf(b1), padf(g1), padf(be1), W2p)

    agg2 = _agg(h2p, src, dst, edge_weights)
    out2, h3p = pl.pallas_call(
        _tc_mid_body,
        out_shape=[jax.ShapeDtypeStruct((N, HP), jnp.float32),
                   jax.ShapeDtypeStruct((N, HP), jnp.float32)],
    )(agg2[0], agg2[1], h2p, dinv, padf(b2), padf(g2), padf(be2), W3p)

    agg3 = _agg(h3p, src, dst, edge_weights)
    y = pl.pallas_call(
        _tc_head_body,
        out_shape=jax.ShapeDtypeStruct((N, C), jnp.float32),
    )(agg3[0], agg3[1], h3p, dinv, padf(b3), out1, out2, Wl1, Wl2, Wl3,
      bl.reshape(1, C))
    return y


# speedup vs baseline: 47.4489x; 1.1139x over previous
import functools

import jax
import jax.numpy as jnp
from jax import lax
from jax.experimental import pallas as pl
from jax.experimental.pallas import tpu as pltpu
from jax.experimental.pallas import tpu_sc as plsc

N = 10000
E = 320000
F_IN = 128
H = 20
HP = 32
C = 10
NC = 2
NS = 16
RPT = 624
TAIL0 = RPT * NS
TAIL = N - TAIL0
CK = 80
EPT = E // (NC * NS)
NCK = EPT // CK

_mesh = plsc.VectorSubcoreMesh(core_axis_name="c", subcore_axis_name="s")
_sc_params = pltpu.CompilerParams(use_tc_tiling_on_sc=False)


def _refill(dstv, dst_flat, j):
    for k0 in range(0, CK, 16):
        dstv[pl.ds(k0, 16)] = dst_flat[pl.ds(j * CK + k0, 16)]


@functools.partial(
    pl.kernel,
    out_type=[jax.ShapeDtypeStruct((N,), jnp.float32),
              jax.ShapeDtypeStruct((N,), jnp.float32)],
    mesh=_mesh,
    scratch_types=[
        pltpu.VMEM_SHARED((N,), jnp.float32),
        pltpu.VMEM((EPT,), jnp.int32),
        pltpu.VMEM((EPT,), jnp.float32),
        pltpu.VMEM((CK,), jnp.int32),
        pltpu.VMEM((640,), jnp.float32),
    ],
    compiler_params=_sc_params,
)
def _deg(ei_hbm, ew_hbm, out0_hbm, out1_hbm, acc, dst_flat, ew_flat, dstv, zb):
    c = lax.axis_index("c")
    s = lax.axis_index("s")
    r0 = s * RPT
    e0 = c * (E // NC) + s * EPT
    pltpu.sync_copy(ei_hbm.at[pl.ds(E + e0, EPT)], dst_flat)
    pltpu.sync_copy(ew_hbm.at[pl.ds(e0, EPT)], ew_flat)

    def zb16(i, _):
        zb[pl.ds(i * 16, 16)] = jnp.zeros((16,), jnp.float32)
        return 0
    lax.fori_loop(0, 640 // 16, zb16, 0)
    pltpu.sync_copy(zb.at[pl.ds(0, RPT)], acc.at[pl.ds(r0, RPT)])

    @pl.when(s == NS - 1)
    def _():
        pltpu.sync_copy(zb.at[pl.ds(0, TAIL)], acc.at[pl.ds(TAIL0, TAIL)])

    plsc.subcore_barrier()

    def chunk(j, _):
        _refill(dstv, dst_flat, j)
        pltpu.sync_copy(ew_flat.at[pl.ds(j * CK, CK)], acc.at[dstv], add=True)
        return 0

    lax.fori_loop(0, NCK, chunk, 0)
    plsc.subcore_barrier()
    pltpu.sync_copy(acc.at[pl.ds(r0, RPT)], zb.at[pl.ds(0, RPT)])

    @pl.when(c == 0)
    def _():
        pltpu.sync_copy(zb.at[pl.ds(0, RPT)], out0_hbm.at[pl.ds(r0, RPT)])

    @pl.when(c == 1)
    def _():
        pltpu.sync_copy(zb.at[pl.ds(0, RPT)], out1_hbm.at[pl.ds(r0, RPT)])

    @pl.when(s == NS - 1)
    def _():
        pltpu.sync_copy(acc.at[pl.ds(TAIL0, TAIL)], zb.at[pl.ds(0, TAIL)])

        @pl.when(c == 0)
        def _():
            pltpu.sync_copy(zb.at[pl.ds(0, TAIL)], out0_hbm.at[pl.ds(TAIL0, TAIL)])

        @pl.when(c == 1)
        def _():
            pltpu.sync_copy(zb.at[pl.ds(0, TAIL)], out1_hbm.at[pl.ds(TAIL0, TAIL)])


@functools.partial(
    pl.kernel,
    out_type=[jax.ShapeDtypeStruct((N, HP), jnp.float32),
              jax.ShapeDtypeStruct((N, HP), jnp.float32)],
    mesh=_mesh,
    scratch_types=[
        pltpu.VMEM_SHARED((N, HP), jnp.float32),
        pltpu.VMEM_SHARED((N, HP), jnp.float32),
        pltpu.VMEM((EPT,), jnp.int32),
        pltpu.VMEM((EPT,), jnp.int32),
        pltpu.VMEM((EPT,), jnp.float32),
        pltpu.VMEM((640,), jnp.float32),
        pltpu.VMEM((CK, HP), jnp.float32),
        pltpu.VMEM((CK, HP), jnp.float32),
        pltpu.VMEM((CK, HP), jnp.float32),
        pltpu.VMEM((CK,), jnp.int32),
        pltpu.VMEM((CK,), jnp.int32),
        pltpu.VMEM((CK,), jnp.int32),
        pltpu.VMEM((640, HP), jnp.float32),
        pltpu.SemaphoreType.DMA,
        pltpu.SemaphoreType.DMA,
        pltpu.SemaphoreType.DMA,
        pltpu.SemaphoreType.DMA,
        pltpu.SemaphoreType.DMA,
        pltpu.SemaphoreType.DMA,
    ],
    compiler_params=_sc_params,
)
def _agg(h_hbm, ei_hbm, ew_hbm, dinv_hbm, out0_hbm, out1_hbm,
         hS, acc, src_flat, dst_flat, ew_flat, dv_t,
         rows_0, rows_1, rows_2, dstv_0, dstv_1, dstv_2, bounce,
         gsem_0, gsem_1, gsem_2, ssem_0, ssem_1, ssem_2):
    c = lax.axis_index("c")
    s = lax.axis_index("s")
    r0 = s * RPT
    e0 = c * (E // NC) + s * EPT
    pltpu.sync_copy(ei_hbm.at[pl.ds(e0, EPT)], src_flat)
    pltpu.sync_copy(ei_hbm.at[pl.ds(E + e0, EPT)], dst_flat)
    pltpu.sync_copy(ew_hbm.at[pl.ds(e0, EPT)], ew_flat)
    pltpu.sync_copy(h_hbm.at[pl.ds(r0, RPT)], bounce.at[pl.ds(0, RPT)])
    pltpu.sync_copy(dinv_hbm.at[pl.ds(r0, RPT)], dv_t.at[pl.ds(0, RPT)])

    @pl.when(s == NS - 1)
    def _():
        pltpu.sync_copy(h_hbm.at[pl.ds(TAIL0, TAIL)], bounce.at[pl.ds(RPT, TAIL)])
        pltpu.sync_copy(dinv_hbm.at[pl.ds(TAIL0, TAIL)], dv_t.at[pl.ds(RPT, TAIL)])

    nrows16 = lax.select(s == NS - 1, 640 // 16, RPT // 16)

    def rowscale(g, _):
        f = dv_t[pl.ds(g * 16, 16)]
        for l in range(16):
            w = f[l]
            k = g * 16 + l
            bounce[k, 0:16] = bounce[k, 0:16] * w
            bounce[k, 16:32] = bounce[k, 16:32] * w
        return 0
    lax.fori_loop(0, nrows16, rowscale, 0)

    pltpu.sync_copy(bounce.at[pl.ds(0, RPT)], hS.at[pl.ds(r0, RPT)])

    @pl.when(s == NS - 1)
    def _():
        pltpu.sync_copy(bounce.at[pl.ds(RPT, TAIL)], hS.at[pl.ds(TAIL0, TAIL)])

    @pl.when(c == 0)
    def _():
        pltpu.sync_copy(bounce.at[pl.ds(0, RPT)], acc.at[pl.ds(r0, RPT)])

        @pl.when(s == NS - 1)
        def _():
            pltpu.sync_copy(bounce.at[pl.ds(RPT, TAIL)], acc.at[pl.ds(TAIL0, TAIL)])

    @pl.when(c == 1)
    def _():
        def zr(i, _):
            rows_0[i, 0:16] = jnp.zeros((16,), jnp.float32)
            rows_0[i, 16:32] = jnp.zeros((16,), jnp.float32)
            return 0
        lax.fori_loop(0, CK, zr, 0)

        def zc(i, _):
            pltpu.sync_copy(rows_0, acc.at[pl.ds(r0 + i * CK, CK)])
            return 0
        lax.fori_loop(0, RPT // CK, zc, 0)
        pltpu.sync_copy(rows_0.at[pl.ds(0, RPT % CK)],
                        acc.at[pl.ds(r0 + CK * (RPT // CK), RPT % CK)])

        @pl.when(s == NS - 1)
        def _():
            pltpu.sync_copy(rows_0.at[pl.ds(0, TAIL)], acc.at[pl.ds(TAIL0, TAIL)])

    plsc.subcore_barrier()

    bufs = ((rows_0, dstv_0, gsem_0, ssem_0),
            (rows_1, dstv_1, gsem_1, ssem_1),
            (rows_2, dstv_2, gsem_2, ssem_2))

    def gather(j, rows, sem):
        pltpu.async_copy(hS.at[src_flat.at[pl.ds(j * CK, CK)]], rows, sem)

    def gather_wait(j, rows, sem):
        pltpu.make_async_copy(hS.at[src_flat.at[pl.ds(j * CK, CK)]], rows, sem).wait()

    def scatter_wait(rows, dstv, sem):
        pltpu.make_async_copy(rows, acc.at[dstv], sem).wait()

    def scale(rows, j):
        def grp(g, _):
            k0 = g * 16
            nv = ew_flat[pl.ds(j * CK + k0, 16)]
            for l in range(16):
                w = nv[l]
                rows[k0 + l, 0:16] = rows[k0 + l, 0:16] * w
                rows[k0 + l, 16:32] = rows[k0 + l, 16:32] * w
            return 0
        lax.fori_loop(0, CK // 16, grp, 0)

    def step(j, u, first=False, guard_next=False):
        nu = (u + 1) % 3
        rows, dstv, gsem, ssem = bufs[u]
        nrows, ndstv, ngsem, nssem = bufs[nu]
        if not first:
            scatter_wait(nrows, ndstv, nssem)
        if guard_next:
            @pl.when(j < NCK - 1)
            def _():
                gather(j + 1, nrows, ngsem)
        else:
            gather(j + 1, nrows, ngsem)
        gather_wait(j, rows, gsem)
        scale(rows, j)
        _refill(dstv, dst_flat, j)
        pltpu.async_copy(rows, acc.at[dstv], ssem, add=True)

    gather(0, rows_0, gsem_0)
    step(0, 0, first=True)
    step(1, 1, first=True)

    def tri(t, _):
        j = 3 * t + 2
        step(j, 2, guard_next=True)
        step(j + 1, 0, guard_next=True)
        step(j + 2, 1, guard_next=True)
        return 0

    lax.fori_loop(0, (NCK - 2) // 3, tri, 0)
    scatter_wait(rows_0, dstv_0, ssem_0)
    scatter_wait(rows_1, dstv_1, ssem_1)

    plsc.subcore_barrier()
    pltpu.sync_copy(acc.at[pl.ds(r0, RPT)], bounce.at[pl.ds(0, RPT)])

    @pl.when(s == NS - 1)
    def _():
        pltpu.sync_copy(acc.at[pl.ds(TAIL0, TAIL)], bounce.at[pl.ds(RPT, TAIL)])

    def outscale(g, _):
        f = dv_t[pl.ds(g * 16, 16)]
        for l in range(16):
            w = f[l]
            k = g * 16 + l
            bounce[k, 0:16] = bounce[k, 0:16] * w
            bounce[k, 16:32] = bounce[k, 16:32] * w
        return 0
    lax.fori_loop(0, nrows16, outscale, 0)

    @pl.when(c == 0)
    def _():
        pltpu.sync_copy(bounce.at[pl.ds(0, RPT)], out0_hbm.at[pl.ds(r0, RPT)])

    @pl.when(c == 1)
    def _():
        pltpu.sync_copy(bounce.at[pl.ds(0, RPT)], out1_hbm.at[pl.ds(r0, RPT)])

    @pl.when(s == NS - 1)
    def _():
        @pl.when(c == 0)
        def _():
            pltpu.sync_copy(bounce.at[pl.ds(RPT, TAIL)],
                            out0_hbm.at[pl.ds(TAIL0, TAIL)])

        @pl.when(c == 1)
        def _():
            pltpu.sync_copy(bounce.at[pl.ds(RPT, TAIL)],
                            out1_hbm.at[pl.ds(TAIL0, TAIL)])


def _pad_vec(v, width):
    return jnp.concatenate([v, jnp.zeros((width - v.shape[0],), v.dtype)])[None, :]


def _pad_mat(w, rows, cols):
    r, c = w.shape
    if rows > r:
        w = jnp.concatenate([w, jnp.zeros((rows - r, c), w.dtype)], axis=0)
    if cols > c:
        w = jnp.concatenate([w, jnp.zeros((rows, cols - c), w.dtype)], axis=1)
    return w


def _tc1_body(x_ref, w1_ref, deg0_ref, deg1_ref, h1_ref, dinv_ref):
    w = _pad_mat(w1_ref[...], F_IN, HP)
    h1_ref[...] = jnp.dot(x_ref[...], w, preferred_element_type=jnp.float32)
    dinv_ref[...] = lax.rsqrt(deg0_ref[...] + deg1_ref[...] + 1.0)


def _tc_mid_body(agga_ref, aggb_ref, b_ref, g_ref, be_ref, wn_ref,
                 bn_ref, hn_ref):
    pre = agga_ref[...] + aggb_ref[...] + _pad_vec(b_ref[...], HP)
    r = jnp.maximum(pre, 0.0)
    mu = jnp.mean(r, axis=0, keepdims=True)
    var = jnp.mean((r - mu) * (r - mu), axis=0, keepdims=True)
    bn = ((r - mu) / jnp.sqrt(var + 1e-5) * _pad_vec(g_ref[...], HP)
          + _pad_vec(be_ref[...], HP))
    bn_ref[...] = bn
    wn = _pad_mat(wn_ref[...], HP, HP)
    hn_ref[...] = jnp.dot(bn, wn, preferred_element_type=jnp.float32)


def _tc_head_body(agga_ref, aggb_ref, b_ref, o1_ref, o2_ref, wl_ref, bl_ref,
                  y_ref):
    pre = agga_ref[...] + aggb_ref[...] + _pad_vec(b_ref[...], HP)
    out3 = jnp.maximum(pre, 0.0)
    wl = wl_ref[...]
    wl1 = _pad_mat(wl[0:H], HP, C)
    wl2 = _pad_mat(wl[H:2 * H], HP, C)
    wl3 = _pad_mat(wl[2 * H:3 * H], HP, C)
    y = (jnp.dot(o1_ref[...], wl1, preferred_element_type=jnp.float32)
         + jnp.dot(o2_ref[...], wl2, preferred_element_type=jnp.float32)
         + jnp.dot(out3, wl3, preferred_element_type=jnp.float32))
    y_ref[...] = y + bl_ref[...][None, :]


def kernel(x, edge_index, edge_weights, W1, b1, g1, be1, W2, b2, g2, be2,
           W3, b3, Wl, bl):
    ei_flat = edge_index.reshape(2 * E)
    deg0, deg1 = _deg(ei_flat, edge_weights)

    h1, dinv = pl.pallas_call(
        _tc1_body,
        out_shape=[jax.ShapeDtypeStruct((N, HP), jnp.float32),
                   jax.ShapeDtypeStruct((N,), jnp.float32)],
    )(x, W1, deg0, deg1)

    agg1 = _agg(h1, ei_flat, edge_weights, dinv)
    out1, h2 = pl.pallas_call(
        _tc_mid_body,
        out_shape=[jax.ShapeDtypeStruct((N, HP), jnp.float32),
                   jax.ShapeDtypeStruct((N, HP), jnp.float32)],
    )(agg1[0], agg1[1], b1, g1, be1, W2)

    agg2 = _agg(h2, ei_flat, edge_weights, dinv)
    out2, h3 = pl.pallas_call(
        _tc_mid_body,
        out_shape=[jax.ShapeDtypeStruct((N, HP), jnp.float32),
                   jax.ShapeDtypeStruct((N, HP), jnp.float32)],
    )(agg2[0], agg2[1], b2, g2, be2, W3)

    agg3 = _agg(h3, ei_flat, edge_weights, dinv)
    y = pl.pallas_call(
        _tc_head_body,
        out_shape=jax.ShapeDtypeStruct((N, C), jnp.float32),
    )(agg3[0], agg3[1], b3, out1, out2, Wl, bl)
    return y
